# baseline (speedup 1.0000x reference)
import functools

import jax
import jax.numpy as jnp
from jax import lax
from jax.experimental import pallas as pl
from jax.experimental.pallas import tpu as pltpu
from jax.experimental.pallas import tpu_sc as plsc

NUM_STATES = 1000000
EMBED_DIM = 32
BATCH = 16384
SEQ_LEN = 200

TOTAL = BATCH * SEQ_LEN
NW = 32
CHUNK = 512
BBLK = CHUNK // 128
CHUNKS_PER_S = BATCH // CHUNK
NCHUNK = TOTAL // (CHUNK * NW)
NBUF = 2


def _emb_body(ids_hbm, table_hbm, out_hbm, idx_v, rows_v, trans_v,
              sem_g0, sem_g1, sem_s0, sem_s1):
    wid = lax.axis_index("s") * 2 + lax.axis_index("c")
    k0 = wid * NCHUNK

    sems_g = (sem_g0, sem_g1)
    sems_s = (sem_s0, sem_s1)
    lanes = lax.iota(jnp.int32, 16)

    def chunk_coords(g):
        k = k0 + g
        s = k // CHUNKS_PER_S
        b0 = (k % CHUNKS_PER_S) * CHUNK
        return s, b0

    def fire(g, b):
        s, b0 = chunk_coords(g)
        pltpu.sync_copy(ids_hbm.at[pl.ds(s * BATCH + b0, CHUNK)],
                        idx_v.at[b])
        pltpu.async_copy(table_hbm.at[idx_v.at[b]],
                         rows_v.at[b], sems_g[b])

    def wait_gather(b):
        pltpu.make_async_copy(
            table_hbm.at[idx_v.at[b]],
            rows_v.at[b], sems_g[b]).wait()

    def transpose(b):
        def col(c, _):
            cb = c // 8
            cr = c % 8
            cvec = jnp.full((16,), c, jnp.int32)
            for bb in range(BBLK):
                for k in range(8):
                    bvec = bb * 128 + k * 16 + lanes
                    val = plsc.load_gather(rows_v.at[b], [bvec, cvec])
                    trans_v[b, cb, bb, cr, pl.ds(k * 16, 16)] = val
            return 0
        lax.fori_loop(0, EMBED_DIM, col, 0)

    def fire_store(g, b):
        s, b0 = chunk_coords(g)
        pltpu.async_copy(trans_v.at[b],
                         out_hbm.at[s, :, pl.ds(b0 // 128, BBLK)],
                         sems_s[b])

    def wait_store(b):
        pltpu.make_async_copy(
            trans_v.at[b], out_hbm.at[0, :, pl.ds(0, BBLK)],
            sems_s[b]).wait()

    fire(0, 0)

    def body(i, _):
        g = 2 * i

        @pl.when(i > 0)
        def _():
            wait_store(1)
        fire(g + 1, 1)

        wait_gather(0)

        @pl.when(i > 0)
        def _():
            wait_store(0)
        transpose(0)
        fire_store(g, 0)

        @pl.when(i < NCHUNK // 2 - 1)
        def _():
            fire(g + 2, 0)

        wait_gather(1)
        transpose(1)
        fire_store(g + 1, 1)
        return 0

    lax.fori_loop(0, NCHUNK // 2, body, 0)

    wait_store(0)
    wait_store(1)


@functools.partial(
    pl.kernel,
    out_type=jax.ShapeDtypeStruct((SEQ_LEN, 4, BATCH // 128, 8, 128),
                                  jnp.float32),
    mesh=plsc.VectorSubcoreMesh(core_axis_name="c", subcore_axis_name="s"),
    scratch_types=[
        pltpu.VMEM((NBUF, CHUNK), jnp.int32),
        pltpu.VMEM((NBUF, CHUNK, EMBED_DIM), jnp.float32),
        pltpu.VMEM((NBUF, 4, BBLK, 8, 128), jnp.float32),
        pltpu.SemaphoreType.DMA,
        pltpu.SemaphoreType.DMA,
        pltpu.SemaphoreType.DMA,
        pltpu.SemaphoreType.DMA,
    ],
    compiler_params=pltpu.CompilerParams(use_tc_tiling_on_sc=False,
                                         needs_layout_passes=False),
)
def _emb_lookup(ids_hbm, table_hbm, out_hbm, idx_v, rows_v, trans_v,
                sem_g0, sem_g1, sem_s0, sem_s1):
    _emb_body(ids_hbm, table_hbm, out_hbm, idx_v, rows_v, trans_v,
              sem_g0, sem_g1, sem_s0, sem_s1)


def kernel(state_ids, table):
    ids = state_ids.astype(jnp.int32).T.reshape(TOTAL)
    out5 = _emb_lookup(ids, table)
    return out5.transpose(2, 4, 0, 1, 3).reshape(BATCH, SEQ_LEN, EMBED_DIM)


# speedup vs baseline: 3.2567x; 3.2567x over previous
import functools

import jax
import jax.numpy as jnp
from jax import lax
from jax.experimental import pallas as pl
from jax.experimental.pallas import tpu as pltpu
from jax.experimental.pallas import tpu_sc as plsc

NUM_STATES = 1000000
EMBED_DIM = 32
BATCH = 16384
SEQ_LEN = 200

TOTAL = BATCH * SEQ_LEN
NW = 32
CHUNK = 512
BBLK = CHUNK // 128
CHUNKS_PER_S = BATCH // CHUNK
NCHUNK = TOTAL // (CHUNK * NW)
NBUF = 2


def _emb_body(ids_hbm, table_hbm, out_hbm, idx_v, rows_v, trans_v,
              sem_g0, sem_g1, sem_s0, sem_s1):
    wid = lax.axis_index("s") * 2 + lax.axis_index("c")
    k0 = wid * NCHUNK

    sems_g = (sem_g0, sem_g1)
    sems_s = (sem_s0, sem_s1)
    lanes = lax.iota(jnp.int32, 16)

    def chunk_coords(g):
        k = k0 + g
        s = k // CHUNKS_PER_S
        b0 = (k % CHUNKS_PER_S) * CHUNK
        return s, b0

    def fire(g, b):
        s, b0 = chunk_coords(g)
        pltpu.sync_copy(ids_hbm.at[pl.ds(s * BATCH + b0, CHUNK)],
                        idx_v.at[b])
        pltpu.async_copy(table_hbm.at[idx_v.at[b]],
                         rows_v.at[b], sems_g[b])

    def wait_gather(b):
        pltpu.make_async_copy(
            table_hbm.at[idx_v.at[b]],
            rows_v.at[b], sems_g[b]).wait()

    cbv = {c0: (c0 + lanes) // 8 for c0 in (0, 16)}
    crv = {c0: (c0 + lanes) % 8 for c0 in (0, 16)}

    def transpose(b):
        def row(bi):
            bbv = jnp.full((16,), bi // 128, jnp.int32)
            bcv = jnp.full((16,), bi % 128, jnp.int32)
            for c0 in (0, 16):
                val = rows_v[b, bi, pl.ds(c0, 16)]
                plsc.store_scatter(trans_v.at[b],
                                   [cbv[c0], bbv, crv[c0], bcv], val)
        plsc.parallel_loop(0, CHUNK, 1, unroll=8)(row)

    def fire_store(g, b):
        s, b0 = chunk_coords(g)
        pltpu.async_copy(trans_v.at[b, :, pl.ds(0, BBLK), :, pl.ds(0, 128)],
                         out_hbm.at[s, :, pl.ds(b0 // 128, BBLK)],
                         sems_s[b])

    def wait_store(b):
        pltpu.make_async_copy(
            trans_v.at[b, :, pl.ds(0, BBLK), :, pl.ds(0, 128)],
            out_hbm.at[0, :, pl.ds(0, BBLK)],
            sems_s[b]).wait()

    fire(0, 0)

    def body(i, _):
        g = 2 * i

        @pl.when(i > 0)
        def _():
            wait_store(1)
        fire(g + 1, 1)

        wait_gather(0)

        @pl.when(i > 0)
        def _():
            wait_store(0)
        transpose(0)
        fire_store(g, 0)

        @pl.when(i < NCHUNK // 2 - 1)
        def _():
            fire(g + 2, 0)

        wait_gather(1)
        transpose(1)
        fire_store(g + 1, 1)
        return 0

    lax.fori_loop(0, NCHUNK // 2, body, 0)

    wait_store(0)
    wait_store(1)


@functools.partial(
    pl.kernel,
    out_type=jax.ShapeDtypeStruct((SEQ_LEN, 4, BATCH // 128, 8, 128),
                                  jnp.float32),
    mesh=plsc.VectorSubcoreMesh(core_axis_name="c", subcore_axis_name="s"),
    scratch_types=[
        pltpu.VMEM((NBUF, CHUNK), jnp.int32),
        pltpu.VMEM((NBUF, CHUNK, EMBED_DIM), jnp.float32),
        pltpu.VMEM((NBUF, 4, BBLK + 1, 8, 129), jnp.float32),
        pltpu.SemaphoreType.DMA,
        pltpu.SemaphoreType.DMA,
        pltpu.SemaphoreType.DMA,
        pltpu.SemaphoreType.DMA,
    ],
    compiler_params=pltpu.CompilerParams(use_tc_tiling_on_sc=False,
                                         needs_layout_passes=False),
)
def _emb_lookup(ids_hbm, table_hbm, out_hbm, idx_v, rows_v, trans_v,
                sem_g0, sem_g1, sem_s0, sem_s1):
    _emb_body(ids_hbm, table_hbm, out_hbm, idx_v, rows_v, trans_v,
              sem_g0, sem_g1, sem_s0, sem_s1)


def kernel(state_ids, table):
    ids = state_ids.astype(jnp.int32).T.reshape(TOTAL)
    out5 = _emb_lookup(ids, table)
    return out5.transpose(2, 4, 0, 1, 3).reshape(BATCH, SEQ_LEN, EMBED_DIM)


# speedup vs baseline: 3.2769x; 1.0062x over previous
import functools

import jax
import jax.numpy as jnp
from jax import lax
from jax.experimental import pallas as pl
from jax.experimental.pallas import tpu as pltpu
from jax.experimental.pallas import tpu_sc as plsc

NUM_STATES = 1000000
EMBED_DIM = 32
BATCH = 16384
SEQ_LEN = 200

TOTAL = BATCH * SEQ_LEN
NW = 32
CHUNK = 512
BBLK = CHUNK // 128
CHUNKS_PER_S = BATCH // CHUNK
NCHUNK = TOTAL // (CHUNK * NW)
NBUF = 2


def _emb_body(ids_hbm, table_hbm, out_hbm, idx_v, rows_v, trans_v,
              sem_g0, sem_g1, sem_s0, sem_s1):
    wid = lax.axis_index("s") * 2 + lax.axis_index("c")
    k0 = wid * NCHUNK

    sems_g = (sem_g0, sem_g1)
    sems_s = (sem_s0, sem_s1)
    lanes = lax.iota(jnp.int32, 16)

    def chunk_coords(g):
        k = k0 + g
        s = k // CHUNKS_PER_S
        b0 = (k % CHUNKS_PER_S) * CHUNK
        return s, b0

    def fire(g, b):
        s, b0 = chunk_coords(g)
        pltpu.sync_copy(ids_hbm.at[pl.ds(s * BATCH + b0, CHUNK)],
                        idx_v.at[b])
        pltpu.async_copy(table_hbm.at[idx_v.at[b]],
                         rows_v.at[b], sems_g[b])

    def wait_gather(b):
        pltpu.make_async_copy(
            table_hbm.at[idx_v.at[b]],
            rows_v.at[b], sems_g[b]).wait()

    cbv = {c0: (c0 + lanes) // 8 for c0 in (0, 16)}
    crv = {c0: (c0 + lanes) % 8 for c0 in (0, 16)}

    def transpose(b):
        def row(bi):
            bbv = jnp.full((16,), bi // 128, jnp.int32)
            bcv = jnp.full((16,), bi % 128, jnp.int32)
            for c0 in (0, 16):
                val = rows_v[b, bi, pl.ds(c0, 16)]
                plsc.store_scatter(trans_v.at[b],
                                   [cbv[c0], bbv, crv[c0], bcv], val)
        plsc.parallel_loop(0, CHUNK, 1, unroll=8)(row)

    def fire_store(g, b):
        s, b0 = chunk_coords(g)
        pltpu.async_copy(trans_v.at[b, :, pl.ds(0, BBLK), :, pl.ds(0, 128)],
                         out_hbm.at[s, :, pl.ds(b0 // 128, BBLK)],
                         sems_s[b])

    def wait_store(b):
        pltpu.make_async_copy(
            trans_v.at[b, :, pl.ds(0, BBLK), :, pl.ds(0, 128)],
            out_hbm.at[0, :, pl.ds(0, BBLK)],
            sems_s[b]).wait()

    fire(0, 0)

    def body(i, _):
        g = 2 * i

        @pl.when(i > 0)
        def _():
            wait_store(1)
        fire(g + 1, 1)

        wait_gather(0)

        @pl.when(i > 0)
        def _():
            wait_store(0)
        transpose(0)
        fire_store(g, 0)

        @pl.when(i < NCHUNK // 2 - 1)
        def _():
            fire(g + 2, 0)

        wait_gather(1)
        transpose(1)
        fire_store(g + 1, 1)
        return 0

    lax.fori_loop(0, NCHUNK // 2, body, 0)

    wait_store(0)
    wait_store(1)


@functools.partial(
    pl.kernel,
    out_type=jax.ShapeDtypeStruct((SEQ_LEN, 4, BATCH // 128, 8, 128),
                                  jnp.float32),
    mesh=plsc.VectorSubcoreMesh(core_axis_name="c", subcore_axis_name="s"),
    scratch_types=[
        pltpu.VMEM((NBUF, CHUNK), jnp.int32),
        pltpu.VMEM((NBUF, CHUNK, EMBED_DIM), jnp.float32),
        pltpu.VMEM((NBUF, 4, BBLK + 1, 8, 129), jnp.float32),
        pltpu.SemaphoreType.DMA,
        pltpu.SemaphoreType.DMA,
        pltpu.SemaphoreType.DMA,
        pltpu.SemaphoreType.DMA,
    ],
    compiler_params=pltpu.CompilerParams(use_tc_tiling_on_sc=False,
                                         needs_layout_passes=False),
)
def _emb_lookup(ids_hbm, table_hbm, out_hbm, idx_v, rows_v, trans_v,
                sem_g0, sem_g1, sem_s0, sem_s1):
    _emb_body(ids_hbm, table_hbm, out_hbm, idx_v, rows_v, trans_v,
              sem_g0, sem_g1, sem_s0, sem_s1)


def kernel(state_ids, table):
    ids = state_ids.astype(jnp.int32).T.reshape(TOTAL) * 4
    t4 = jnp.pad(table, ((0, 0), (0, 96))).reshape(4 * NUM_STATES,
                                                   EMBED_DIM)
    out5 = _emb_lookup(ids, t4)
    return out5.transpose(2, 4, 0, 1, 3).reshape(BATCH, SEQ_LEN, EMBED_DIM)
